# baseline (speedup 1.0000x reference)
import functools

import jax
import jax.numpy as jnp
from jax.experimental import pallas as pl
from jax.experimental.pallas import tpu as pltpu

L, B, NP, NG, S, P, C = 6, 4, 1000, 50, 20, 20, 3
D = P * 2
ALPHA, GAMMA = 0.25, 2.0
W_CLS, W_PTS, W_DIR = 2.0, 5.0, 0.005
COST_CLS, COST_PTS = 2.0, 5.0
EPS = 1e-8


def _problem_kernel(predT_r, gt3_r, gtflat_r, clsT_r, oh_r, out_r, rm_ref):
    pred = predT_r[0, 0]
    big = jnp.float32(3.4e38)

    LC = 500

    def s_body(s, _):
        gts = gt3_r[0, s]
        for lo in range(0, NP, LC):
            hi = min(lo + LC, NP)
            acc = jnp.abs(gts[:, 0:1] - pred[0:1, lo:hi])
            for d in range(1, D):
                gcol = gts[:, d:d + 1]
                prow = pred[d:d + 1, lo:hi]
                acc = acc + jnp.abs(gcol - prow)
            rm_ref[:, lo:hi] = jnp.minimum(rm_ref[:, lo:hi], acc)
        return 0

    rm_ref[...] = jnp.full((NG, NP), big, jnp.float32)
    jax.lax.fori_loop(0, S, s_body, 0)
    runmin = rm_ref[...]

    x = clsT_r[0, 0]
    p = jax.nn.sigmoid(x)
    one_m_p = 1.0 - p
    pos = ALPHA * (one_m_p * one_m_p) * (-jnp.log(p + EPS))
    neg = (1.0 - ALPHA) * (p * p) * (-jnp.log(one_m_p + EPS))
    pn = pos - neg
    oh = oh_r[0]
    cls_T = (oh[:, 0:1] * pn[0:1, :]
             + oh[:, 1:2] * pn[1:2, :]
             + oh[:, 2:3] * pn[2:3, :])

    cost = cls_T * COST_CLS + runmin * COST_PTS

    rowmin = jnp.min(cost, axis=1, keepdims=True)
    l_iota = jax.lax.broadcasted_iota(jnp.int32, (NG, NP), 1)
    argidx = jnp.min(jnp.where(cost == rowmin, l_iota, NP),
                     axis=1, keepdims=True)
    onehot_assign = l_iota == argidx

    ng_iota = jax.lax.broadcasted_iota(jnp.int32, (NG, NP), 0)
    colmax = jnp.max(jnp.where(onehot_assign, ng_iota, -1),
                     axis=0, keepdims=True)
    winner = onehot_assign & (ng_iota == colmax)
    wf = winner.astype(jnp.float32)

    num_pos = jnp.sum(wf)
    sum_pts = jnp.sum(wf * runmin)
    sum_corr = jnp.sum(wf * cls_T)
    bg = jnp.sum(neg)

    roww = jnp.max(wf, axis=1, keepdims=True)
    gpred = jax.lax.dot_general(
        wf, pred, (((1,), (1,)), ((), ())),
        preferred_element_type=jnp.float32)

    srunmin = jnp.full((NG, 1), big, jnp.float32)
    srunarg = jnp.zeros((NG, 1), jnp.int32)
    for s in range(S):
        g = gt3_r[0, s]
        dsum = jnp.sum(jnp.abs(gpred - g), axis=1, keepdims=True)
        better = dsum < srunmin
        srunarg = jnp.where(better, s, srunarg)
        srunmin = jnp.minimum(srunmin, dsum)

    ng_col = jax.lax.broadcasted_iota(jnp.int32, (NG, 1), 0)
    jcol = ng_col * S + srunarg
    j_iota = jax.lax.broadcasted_iota(jnp.int32, (NG, NG * S), 1)
    ohns = (j_iota == jcol).astype(jnp.float32)
    gtd = jax.lax.dot_general(
        ohns, gtflat_r[0], (((1,), (0,)), ((), ())),
        preferred_element_type=jnp.float32)

    pd = gpred[:, 2:] - gpred[:, :-2]
    td = gtd[:, 2:] - gtd[:, :-2]
    r2 = jax.lax.broadcasted_iota(jnp.int32, (D - 2, P - 1), 0) // 2
    c2 = jax.lax.broadcasted_iota(jnp.int32, (D - 2, P - 1), 1)
    pair_m = (r2 == c2).astype(jnp.float32)
    stk = jnp.concatenate([pd * td, pd * pd, td * td], axis=0)
    res = jax.lax.dot_general(stk, pair_m, (((1,), (0,)), ((), ())),
                              preferred_element_type=jnp.float32)
    dots = res[0:NG]
    pdn = res[NG:2 * NG]
    tdn = res[2 * NG:3 * NG]
    cos = dots / (jnp.sqrt(pdn) * jnp.sqrt(tdn) + EPS)
    sum_dir = jnp.sum((1.0 - cos) * roww)

    riota = jax.lax.broadcasted_iota(jnp.int32, (8, 128), 0)
    arr = (jnp.where(riota == 0, num_pos, 0.0)
           + jnp.where(riota == 1, sum_pts, 0.0)
           + jnp.where(riota == 2, sum_corr, 0.0)
           + jnp.where(riota == 3, bg, 0.0)
           + jnp.where(riota == 4, sum_dir, 0.0))
    out_r[0, 0] = arr


@jax.jit
def kernel(all_cls_scores, all_line_preds, gt_labels, gt_shifts_pts):
    predn = all_line_preds.reshape(L, B, NP, D)
    predT = predn.transpose(0, 1, 3, 2)
    gt3 = gt_shifts_pts.reshape(B, NG, S, D).transpose(0, 2, 1, 3)
    gtflat = gt_shifts_pts.reshape(B, NG * S, D)
    clsT = all_cls_scores.transpose(0, 1, 3, 2)
    oh = jax.nn.one_hot(gt_labels, C, dtype=jnp.float32)

    parts = pl.pallas_call(
        _problem_kernel,
        grid=(B, L),
        in_specs=[
            pl.BlockSpec((1, 1, D, NP), lambda b, l: (l, b, 0, 0)),
            pl.BlockSpec((1, S, NG, D), lambda b, l: (b, 0, 0, 0)),
            pl.BlockSpec((1, NG * S, D), lambda b, l: (b, 0, 0)),
            pl.BlockSpec((1, 1, C, NP), lambda b, l: (l, b, 0, 0)),
            pl.BlockSpec((1, NG, C), lambda b, l: (b, 0, 0)),
        ],
        out_specs=pl.BlockSpec((1, 1, 8, 128), lambda b, l: (l, b, 0, 0)),
        out_shape=jax.ShapeDtypeStruct((L, B, 8, 128), jnp.float32),
        scratch_shapes=[pltpu.VMEM((NG, NP), jnp.float32)],
    )(predT, gt3, gtflat, clsT, oh)

    v = parts[:, :, :, 0]
    num_pos = jnp.maximum(v[:, :, 0].sum(axis=1), 1.0)
    loss_cls = (v[:, :, 3].sum(axis=1) + v[:, :, 2].sum(axis=1)) / num_pos * W_CLS
    loss_pts = v[:, :, 1].sum(axis=1) / num_pos * W_PTS
    loss_dir = v[:, :, 4].sum(axis=1) / num_pos * W_DIR
    out = jnp.stack([loss_cls, loss_pts, loss_dir], axis=1)
    return jnp.nan_to_num(out)


# speedup vs baseline: 1.0066x; 1.0066x over previous
import functools

import jax
import jax.numpy as jnp
from jax.experimental import pallas as pl
from jax.experimental.pallas import tpu as pltpu

L, B, NP, NG, S, P, C = 6, 4, 1000, 50, 20, 20, 3
D = P * 2
ALPHA, GAMMA = 0.25, 2.0
W_CLS, W_PTS, W_DIR = 2.0, 5.0, 0.005
COST_CLS, COST_PTS = 2.0, 5.0
EPS = 1e-8


def _problem_kernel(predT_r, gt3_r, gtflat_r, clsT_r, oh_r, out_r, rm_ref):
    pred = predT_r[0, 0]
    big = jnp.float32(3.4e38)

    LC = 512

    def s_body(s, _):
        gts = gt3_r[0, s]
        for lo in range(0, NP, LC):
            hi = min(lo + LC, NP)
            acc = jnp.abs(gts[:, 0:1] - pred[0:1, lo:hi])
            for d in range(1, D):
                gcol = gts[:, d:d + 1]
                prow = pred[d:d + 1, lo:hi]
                acc = acc + jnp.abs(gcol - prow)
            rm_ref[:, lo:hi] = jnp.minimum(rm_ref[:, lo:hi], acc)
        return 0

    rm_ref[...] = jnp.full((NG, NP), big, jnp.float32)
    jax.lax.fori_loop(0, S, s_body, 0)
    runmin = rm_ref[...]

    x = clsT_r[0, 0]
    p = jax.nn.sigmoid(x)
    one_m_p = 1.0 - p
    pos = ALPHA * (one_m_p * one_m_p) * (-jnp.log(p + EPS))
    neg = (1.0 - ALPHA) * (p * p) * (-jnp.log(one_m_p + EPS))
    pn = pos - neg
    oh = oh_r[0]
    cls_T = (oh[:, 0:1] * pn[0:1, :]
             + oh[:, 1:2] * pn[1:2, :]
             + oh[:, 2:3] * pn[2:3, :])

    cost = cls_T * COST_CLS + runmin * COST_PTS

    rowmin = jnp.min(cost, axis=1, keepdims=True)
    l_iota = jax.lax.broadcasted_iota(jnp.int32, (NG, NP), 1)
    argidx = jnp.min(jnp.where(cost == rowmin, l_iota, NP),
                     axis=1, keepdims=True)
    onehot_assign = l_iota == argidx

    ng_iota = jax.lax.broadcasted_iota(jnp.int32, (NG, NP), 0)
    colmax = jnp.max(jnp.where(onehot_assign, ng_iota, -1),
                     axis=0, keepdims=True)
    winner = onehot_assign & (ng_iota == colmax)
    wf = winner.astype(jnp.float32)

    num_pos = jnp.sum(wf)
    sum_pts = jnp.sum(wf * runmin)
    sum_corr = jnp.sum(wf * cls_T)
    bg = jnp.sum(neg)

    roww = jnp.max(wf, axis=1, keepdims=True)
    gpred = jax.lax.dot_general(
        wf, pred, (((1,), (1,)), ((), ())),
        preferred_element_type=jnp.float32)

    srunmin = jnp.full((NG, 1), big, jnp.float32)
    srunarg = jnp.zeros((NG, 1), jnp.int32)
    for s in range(S):
        g = gt3_r[0, s]
        dsum = jnp.sum(jnp.abs(gpred - g), axis=1, keepdims=True)
        better = dsum < srunmin
        srunarg = jnp.where(better, s, srunarg)
        srunmin = jnp.minimum(srunmin, dsum)

    ng_col = jax.lax.broadcasted_iota(jnp.int32, (NG, 1), 0)
    jcol = ng_col * S + srunarg
    j_iota = jax.lax.broadcasted_iota(jnp.int32, (NG, NG * S), 1)
    ohns = (j_iota == jcol).astype(jnp.float32)
    gtd = jax.lax.dot_general(
        ohns, gtflat_r[0], (((1,), (0,)), ((), ())),
        preferred_element_type=jnp.float32)

    pd = gpred[:, 2:] - gpred[:, :-2]
    td = gtd[:, 2:] - gtd[:, :-2]
    r2 = jax.lax.broadcasted_iota(jnp.int32, (D - 2, P - 1), 0) // 2
    c2 = jax.lax.broadcasted_iota(jnp.int32, (D - 2, P - 1), 1)
    pair_m = (r2 == c2).astype(jnp.float32)
    stk = jnp.concatenate([pd * td, pd * pd, td * td], axis=0)
    res = jax.lax.dot_general(stk, pair_m, (((1,), (0,)), ((), ())),
                              preferred_element_type=jnp.float32)
    dots = res[0:NG]
    pdn = res[NG:2 * NG]
    tdn = res[2 * NG:3 * NG]
    cos = dots / (jnp.sqrt(pdn) * jnp.sqrt(tdn) + EPS)
    sum_dir = jnp.sum((1.0 - cos) * roww)

    riota = jax.lax.broadcasted_iota(jnp.int32, (8, 128), 0)
    arr = (jnp.where(riota == 0, num_pos, 0.0)
           + jnp.where(riota == 1, sum_pts, 0.0)
           + jnp.where(riota == 2, sum_corr, 0.0)
           + jnp.where(riota == 3, bg, 0.0)
           + jnp.where(riota == 4, sum_dir, 0.0))
    out_r[0, 0] = arr


@jax.jit
def kernel(all_cls_scores, all_line_preds, gt_labels, gt_shifts_pts):
    predn = all_line_preds.reshape(L, B, NP, D)
    predT = predn.transpose(0, 1, 3, 2)
    gt3 = gt_shifts_pts.reshape(B, NG, S, D).transpose(0, 2, 1, 3)
    gtflat = gt_shifts_pts.reshape(B, NG * S, D)
    clsT = all_cls_scores.transpose(0, 1, 3, 2)
    oh = jax.nn.one_hot(gt_labels, C, dtype=jnp.float32)

    parts = pl.pallas_call(
        _problem_kernel,
        grid=(B, L),
        in_specs=[
            pl.BlockSpec((1, 1, D, NP), lambda b, l: (l, b, 0, 0)),
            pl.BlockSpec((1, S, NG, D), lambda b, l: (b, 0, 0, 0)),
            pl.BlockSpec((1, NG * S, D), lambda b, l: (b, 0, 0)),
            pl.BlockSpec((1, 1, C, NP), lambda b, l: (l, b, 0, 0)),
            pl.BlockSpec((1, NG, C), lambda b, l: (b, 0, 0)),
        ],
        out_specs=pl.BlockSpec((1, 1, 8, 128), lambda b, l: (l, b, 0, 0)),
        out_shape=jax.ShapeDtypeStruct((L, B, 8, 128), jnp.float32),
        scratch_shapes=[pltpu.VMEM((NG, NP), jnp.float32)],
    )(predT, gt3, gtflat, clsT, oh)

    v = parts[:, :, :, 0]
    num_pos = jnp.maximum(v[:, :, 0].sum(axis=1), 1.0)
    loss_cls = (v[:, :, 3].sum(axis=1) + v[:, :, 2].sum(axis=1)) / num_pos * W_CLS
    loss_pts = v[:, :, 1].sum(axis=1) / num_pos * W_PTS
    loss_dir = v[:, :, 4].sum(axis=1) / num_pos * W_DIR
    out = jnp.stack([loss_cls, loss_pts, loss_dir], axis=1)
    return jnp.nan_to_num(out)


# speedup vs baseline: 1.0492x; 1.0423x over previous
import functools

import jax
import jax.numpy as jnp
from jax.experimental import pallas as pl
from jax.experimental.pallas import tpu as pltpu

L, B, NP, NG, S, P, C = 6, 4, 1000, 50, 20, 20, 3
D = P * 2
ALPHA, GAMMA = 0.25, 2.0
W_CLS, W_PTS, W_DIR = 2.0, 5.0, 0.005
COST_CLS, COST_PTS = 2.0, 5.0
EPS = 1e-8


def _problem_kernel(predT_r, gt3_r, gtflat_r, clsT_r, oh_r, out_r, rm_ref):
    pred = predT_r[0, 0]
    big = jnp.float32(3.4e38)

    LC = 256

    def s_step(s, first):
        gts = gt3_r[0, s]
        for lo in range(0, NP, LC):
            hi = min(lo + LC, NP)
            acc = jnp.abs(gts[:, 0:1] - pred[0:1, lo:hi])
            for d in range(1, D):
                gcol = gts[:, d:d + 1]
                prow = pred[d:d + 1, lo:hi]
                acc = acc + jnp.abs(gcol - prow)
            if first:
                rm_ref[:, lo:hi] = acc
            else:
                rm_ref[:, lo:hi] = jnp.minimum(rm_ref[:, lo:hi], acc)

    s_step(0, True)

    def s_body(s, _):
        s_step(s, False)
        return 0

    jax.lax.fori_loop(1, S, s_body, 0)
    runmin = rm_ref[...]

    x = clsT_r[0, 0]
    p = jax.nn.sigmoid(x)
    one_m_p = 1.0 - p
    pos = ALPHA * (one_m_p * one_m_p) * (-jnp.log(p + EPS))
    neg = (1.0 - ALPHA) * (p * p) * (-jnp.log(one_m_p + EPS))
    pn = pos - neg
    oh = oh_r[0]
    cls_T = (oh[:, 0:1] * pn[0:1, :]
             + oh[:, 1:2] * pn[1:2, :]
             + oh[:, 2:3] * pn[2:3, :])

    cost = cls_T * COST_CLS + runmin * COST_PTS

    rowmin = jnp.min(cost, axis=1, keepdims=True)
    l_iota = jax.lax.broadcasted_iota(jnp.int32, (NG, NP), 1)
    argidx = jnp.min(jnp.where(cost == rowmin, l_iota, NP),
                     axis=1, keepdims=True)
    onehot_assign = l_iota == argidx

    ng_iota = jax.lax.broadcasted_iota(jnp.int32, (NG, NP), 0)
    colmax = jnp.max(jnp.where(onehot_assign, ng_iota, -1),
                     axis=0, keepdims=True)
    winner = onehot_assign & (ng_iota == colmax)
    wf = winner.astype(jnp.float32)

    num_pos = jnp.sum(wf)
    sum_pts = jnp.sum(wf * runmin)
    sum_corr = jnp.sum(wf * cls_T)
    bg = jnp.sum(neg)

    roww = jnp.max(wf, axis=1, keepdims=True)
    gpred = jax.lax.dot_general(
        wf, pred, (((1,), (1,)), ((), ())),
        preferred_element_type=jnp.float32)

    srunmin = jnp.full((NG, 1), big, jnp.float32)
    srunarg = jnp.zeros((NG, 1), jnp.int32)
    for s in range(S):
        g = gt3_r[0, s]
        dsum = jnp.sum(jnp.abs(gpred - g), axis=1, keepdims=True)
        better = dsum < srunmin
        srunarg = jnp.where(better, s, srunarg)
        srunmin = jnp.minimum(srunmin, dsum)

    ng_col = jax.lax.broadcasted_iota(jnp.int32, (NG, 1), 0)
    jcol = ng_col * S + srunarg
    j_iota = jax.lax.broadcasted_iota(jnp.int32, (NG, NG * S), 1)
    ohns = (j_iota == jcol).astype(jnp.float32)
    gtd = jax.lax.dot_general(
        ohns, gtflat_r[0], (((1,), (0,)), ((), ())),
        preferred_element_type=jnp.float32)

    pd = gpred[:, 2:] - gpred[:, :-2]
    td = gtd[:, 2:] - gtd[:, :-2]
    r2 = jax.lax.broadcasted_iota(jnp.int32, (D - 2, P - 1), 0) // 2
    c2 = jax.lax.broadcasted_iota(jnp.int32, (D - 2, P - 1), 1)
    pair_m = (r2 == c2).astype(jnp.float32)
    stk = jnp.concatenate([pd * td, pd * pd, td * td], axis=0)
    res = jax.lax.dot_general(stk, pair_m, (((1,), (0,)), ((), ())),
                              preferred_element_type=jnp.float32)
    dots = res[0:NG]
    pdn = res[NG:2 * NG]
    tdn = res[2 * NG:3 * NG]
    cos = dots / (jnp.sqrt(pdn) * jnp.sqrt(tdn) + EPS)
    sum_dir = jnp.sum((1.0 - cos) * roww)

    riota = jax.lax.broadcasted_iota(jnp.int32, (8, 128), 0)
    arr = (jnp.where(riota == 0, num_pos, 0.0)
           + jnp.where(riota == 1, sum_pts, 0.0)
           + jnp.where(riota == 2, sum_corr, 0.0)
           + jnp.where(riota == 3, bg, 0.0)
           + jnp.where(riota == 4, sum_dir, 0.0))
    out_r[0, 0] = arr


@jax.jit
def kernel(all_cls_scores, all_line_preds, gt_labels, gt_shifts_pts):
    predn = all_line_preds.reshape(L, B, NP, D)
    predT = predn.transpose(0, 1, 3, 2)
    gt3 = gt_shifts_pts.reshape(B, NG, S, D).transpose(0, 2, 1, 3)
    gtflat = gt_shifts_pts.reshape(B, NG * S, D)
    clsT = all_cls_scores.transpose(0, 1, 3, 2)
    oh = jax.nn.one_hot(gt_labels, C, dtype=jnp.float32)

    parts = pl.pallas_call(
        _problem_kernel,
        grid=(B, L),
        in_specs=[
            pl.BlockSpec((1, 1, D, NP), lambda b, l: (l, b, 0, 0)),
            pl.BlockSpec((1, S, NG, D), lambda b, l: (b, 0, 0, 0)),
            pl.BlockSpec((1, NG * S, D), lambda b, l: (b, 0, 0)),
            pl.BlockSpec((1, 1, C, NP), lambda b, l: (l, b, 0, 0)),
            pl.BlockSpec((1, NG, C), lambda b, l: (b, 0, 0)),
        ],
        out_specs=pl.BlockSpec((1, 1, 8, 128), lambda b, l: (l, b, 0, 0)),
        out_shape=jax.ShapeDtypeStruct((L, B, 8, 128), jnp.float32),
        scratch_shapes=[pltpu.VMEM((NG, NP), jnp.float32)],
    )(predT, gt3, gtflat, clsT, oh)

    v = parts[:, :, :, 0]
    num_pos = jnp.maximum(v[:, :, 0].sum(axis=1), 1.0)
    loss_cls = (v[:, :, 3].sum(axis=1) + v[:, :, 2].sum(axis=1)) / num_pos * W_CLS
    loss_pts = v[:, :, 1].sum(axis=1) / num_pos * W_PTS
    loss_dir = v[:, :, 4].sum(axis=1) / num_pos * W_DIR
    out = jnp.stack([loss_cls, loss_pts, loss_dir], axis=1)
    return jnp.nan_to_num(out)


# speedup vs baseline: 1.0585x; 1.0089x over previous
import functools

import jax
import jax.numpy as jnp
from jax.experimental import pallas as pl
from jax.experimental.pallas import tpu as pltpu

L, B, NP, NG, S, P, C = 6, 4, 1000, 50, 20, 20, 3
D = P * 2
ALPHA, GAMMA = 0.25, 2.0
W_CLS, W_PTS, W_DIR = 2.0, 5.0, 0.005
COST_CLS, COST_PTS = 2.0, 5.0
EPS = 1e-8


def _problem_kernel(predT_r, gt3_r, gtflat_r, clsT_r, oh_r, out_r, rm_ref):
    pred = predT_r[0, 0]
    big = jnp.float32(3.4e38)

    LC = 128

    def s_step(s, first):
        gts = gt3_r[0, s]
        for lo in range(0, NP, LC):
            hi = min(lo + LC, NP)
            acc = jnp.abs(gts[:, 0:1] - pred[0:1, lo:hi])
            for d in range(1, D):
                gcol = gts[:, d:d + 1]
                prow = pred[d:d + 1, lo:hi]
                acc = acc + jnp.abs(gcol - prow)
            if first:
                rm_ref[:, lo:hi] = acc
            else:
                rm_ref[:, lo:hi] = jnp.minimum(rm_ref[:, lo:hi], acc)

    s_step(0, True)

    def s_body(s, _):
        s_step(s, False)
        return 0

    jax.lax.fori_loop(1, S, s_body, 0)
    runmin = rm_ref[...]

    x = clsT_r[0, 0]
    p = jax.nn.sigmoid(x)
    one_m_p = 1.0 - p
    pos = ALPHA * (one_m_p * one_m_p) * (-jnp.log(p + EPS))
    neg = (1.0 - ALPHA) * (p * p) * (-jnp.log(one_m_p + EPS))
    pn = pos - neg
    oh = oh_r[0]
    cls_T = (oh[:, 0:1] * pn[0:1, :]
             + oh[:, 1:2] * pn[1:2, :]
             + oh[:, 2:3] * pn[2:3, :])

    cost = cls_T * COST_CLS + runmin * COST_PTS

    rowmin = jnp.min(cost, axis=1, keepdims=True)
    l_iota = jax.lax.broadcasted_iota(jnp.int32, (NG, NP), 1)
    argidx = jnp.min(jnp.where(cost == rowmin, l_iota, NP),
                     axis=1, keepdims=True)
    onehot_assign = l_iota == argidx

    ng_iota = jax.lax.broadcasted_iota(jnp.int32, (NG, NP), 0)
    colmax = jnp.max(jnp.where(onehot_assign, ng_iota, -1),
                     axis=0, keepdims=True)
    winner = onehot_assign & (ng_iota == colmax)
    wf = winner.astype(jnp.float32)

    num_pos = jnp.sum(wf)
    sum_pts = jnp.sum(wf * runmin)
    sum_corr = jnp.sum(wf * cls_T)
    bg = jnp.sum(neg)

    roww = jnp.max(wf, axis=1, keepdims=True)
    gpred = jax.lax.dot_general(
        wf, pred, (((1,), (1,)), ((), ())),
        preferred_element_type=jnp.float32)

    srunmin = jnp.full((NG, 1), big, jnp.float32)
    srunarg = jnp.zeros((NG, 1), jnp.int32)
    for s in range(S):
        g = gt3_r[0, s]
        dsum = jnp.sum(jnp.abs(gpred - g), axis=1, keepdims=True)
        better = dsum < srunmin
        srunarg = jnp.where(better, s, srunarg)
        srunmin = jnp.minimum(srunmin, dsum)

    ng_col = jax.lax.broadcasted_iota(jnp.int32, (NG, 1), 0)
    jcol = ng_col * S + srunarg
    j_iota = jax.lax.broadcasted_iota(jnp.int32, (NG, NG * S), 1)
    ohns = (j_iota == jcol).astype(jnp.float32)
    gtd = jax.lax.dot_general(
        ohns, gtflat_r[0], (((1,), (0,)), ((), ())),
        preferred_element_type=jnp.float32)

    pd = gpred[:, 2:] - gpred[:, :-2]
    td = gtd[:, 2:] - gtd[:, :-2]
    r2 = jax.lax.broadcasted_iota(jnp.int32, (D - 2, P - 1), 0) // 2
    c2 = jax.lax.broadcasted_iota(jnp.int32, (D - 2, P - 1), 1)
    pair_m = (r2 == c2).astype(jnp.float32)
    stk = jnp.concatenate([pd * td, pd * pd, td * td], axis=0)
    res = jax.lax.dot_general(stk, pair_m, (((1,), (0,)), ((), ())),
                              preferred_element_type=jnp.float32)
    dots = res[0:NG]
    pdn = res[NG:2 * NG]
    tdn = res[2 * NG:3 * NG]
    cos = dots / (jnp.sqrt(pdn) * jnp.sqrt(tdn) + EPS)
    sum_dir = jnp.sum((1.0 - cos) * roww)

    riota = jax.lax.broadcasted_iota(jnp.int32, (8, 128), 0)
    arr = (jnp.where(riota == 0, num_pos, 0.0)
           + jnp.where(riota == 1, sum_pts, 0.0)
           + jnp.where(riota == 2, sum_corr, 0.0)
           + jnp.where(riota == 3, bg, 0.0)
           + jnp.where(riota == 4, sum_dir, 0.0))
    out_r[0, 0] = arr


@jax.jit
def kernel(all_cls_scores, all_line_preds, gt_labels, gt_shifts_pts):
    predn = all_line_preds.reshape(L, B, NP, D)
    predT = predn.transpose(0, 1, 3, 2)
    gt3 = gt_shifts_pts.reshape(B, NG, S, D).transpose(0, 2, 1, 3)
    gtflat = gt_shifts_pts.reshape(B, NG * S, D)
    clsT = all_cls_scores.transpose(0, 1, 3, 2)
    oh = jax.nn.one_hot(gt_labels, C, dtype=jnp.float32)

    parts = pl.pallas_call(
        _problem_kernel,
        grid=(B, L),
        in_specs=[
            pl.BlockSpec((1, 1, D, NP), lambda b, l: (l, b, 0, 0)),
            pl.BlockSpec((1, S, NG, D), lambda b, l: (b, 0, 0, 0)),
            pl.BlockSpec((1, NG * S, D), lambda b, l: (b, 0, 0)),
            pl.BlockSpec((1, 1, C, NP), lambda b, l: (l, b, 0, 0)),
            pl.BlockSpec((1, NG, C), lambda b, l: (b, 0, 0)),
        ],
        out_specs=pl.BlockSpec((1, 1, 8, 128), lambda b, l: (l, b, 0, 0)),
        out_shape=jax.ShapeDtypeStruct((L, B, 8, 128), jnp.float32),
        scratch_shapes=[pltpu.VMEM((NG, NP), jnp.float32)],
    )(predT, gt3, gtflat, clsT, oh)

    v = parts[:, :, :, 0]
    num_pos = jnp.maximum(v[:, :, 0].sum(axis=1), 1.0)
    loss_cls = (v[:, :, 3].sum(axis=1) + v[:, :, 2].sum(axis=1)) / num_pos * W_CLS
    loss_pts = v[:, :, 1].sum(axis=1) / num_pos * W_PTS
    loss_dir = v[:, :, 4].sum(axis=1) / num_pos * W_DIR
    out = jnp.stack([loss_cls, loss_pts, loss_dir], axis=1)
    return jnp.nan_to_num(out)
